# Initial kernel scaffold; baseline (speedup 1.0000x reference)
import jax
import jax.numpy as jnp
from jax.experimental import pallas as pl


def kernel(x, edge_index, curva, idx, W1, b1, W2, b2, Wd1, bd1, Wd2, bd2):
    raise NotImplementedError("write your pallas kernel here")


# speedup vs baseline: 4.5479x; 4.5479x over previous
import functools

import jax
import jax.numpy as jnp
from jax import lax
from jax.experimental import pallas as pl
from jax.experimental.pallas import tpu as pltpu
from jax.experimental.pallas import tpu_sc as plsc

N = 10000
E = 320000
FEAT = 128
B = 16384

NC = 2
NS = 16
NW = NC * NS
C = 128

CH = -(-(E // NW) // C)
if CH % 2:
    CH += 1
EPAD = NW * CH * C

GCH = (2 * B) // (NW * C)
ROWS_PER_TILE = -(-N // (NS * 8)) * 8
NPAD = NS * ROWS_PER_TILE


def _scatter_body(x_hbm, ei_hbm, out_hbm,
                  idx_v, buf0, buf1, acc, sem0, sem1):
    c = lax.axis_index("c")
    s = lax.axis_index("s")
    w = s * NC + c
    pltpu.sync_copy(x_hbm.at[pl.ds(s * ROWS_PER_TILE, ROWS_PER_TILE)],
                    acc.at[pl.ds(s * ROWS_PER_TILE, ROWS_PER_TILE)])
    plsc.subcore_barrier()

    pltpu.sync_copy(ei_hbm.at[w, 0], idx_v.at[0])
    pltpu.async_copy(x_hbm.at[idx_v.at[0, 0]], buf0, sem0)
    pltpu.sync_copy(ei_hbm.at[w, 1], idx_v.at[1])
    pltpu.async_copy(x_hbm.at[idx_v.at[1, 0]], buf1, sem1)

    def step(i, carry):
        g = i * 2
        for b, (buf, sem) in enumerate(((buf0, sem0), (buf1, sem1))):
            j = g + b
            pltpu.make_async_copy(x_hbm.at[idx_v.at[b, 0]], buf, sem).wait()
            pltpu.sync_copy(buf, acc.at[idx_v.at[b, 1]], add=True)

            @pl.when(j + 2 < CH)
            def _():
                pltpu.sync_copy(ei_hbm.at[w, j + 2], idx_v.at[b])
                pltpu.async_copy(x_hbm.at[idx_v.at[b, 0]], buf, sem)
        return carry

    lax.fori_loop(0, CH // 2, step, 0)
    plsc.subcore_barrier()
    pltpu.sync_copy(acc.at[pl.ds(s * ROWS_PER_TILE, ROWS_PER_TILE)],
                    out_hbm.at[c, pl.ds(s * ROWS_PER_TILE, ROWS_PER_TILE)])


def _gather_body(x1_hbm, idx_hbm, out_hbm, idx_v, buf0, buf1, sem0, sem1):
    c = lax.axis_index("c")
    s = lax.axis_index("s")
    w = s * NC + c
    base = w * GCH * C
    pltpu.sync_copy(idx_hbm.at[w], idx_v)
    pltpu.async_copy(x1_hbm.at[idx_v.at[0]], buf0, sem0)
    pltpu.async_copy(x1_hbm.at[idx_v.at[1]], buf1, sem1)

    def step(i, carry):
        g = i * 2
        for b, (buf, sem) in enumerate(((buf0, sem0), (buf1, sem1))):
            j = g + b
            pltpu.make_async_copy(x1_hbm.at[idx_v.at[j]], buf, sem).wait()
            pltpu.sync_copy(buf, out_hbm.at[pl.ds(base + j * C, C)])

            @pl.when(j + 2 < GCH)
            def _():
                pltpu.async_copy(x1_hbm.at[idx_v.at[j + 2]], buf, sem)
        return carry

    lax.fori_loop(0, GCH // 2, step, 0)


_sc_mesh = plsc.VectorSubcoreMesh(core_axis_name="c", subcore_axis_name="s")

_edge_scatter = functools.partial(
    pl.kernel,
    out_type=jax.ShapeDtypeStruct((NC, NPAD, FEAT), jnp.float32),
    mesh=_sc_mesh,
    scratch_types=[
        pltpu.VMEM((2, 2, C), jnp.int32),
        pltpu.VMEM((C, FEAT), jnp.float32),
        pltpu.VMEM((C, FEAT), jnp.float32),
        pltpu.VMEM_SHARED((NPAD + 8, FEAT), jnp.float32),
        pltpu.SemaphoreType.DMA,
        pltpu.SemaphoreType.DMA,
    ],
)(_scatter_body)

_pair_gather = functools.partial(
    pl.kernel,
    out_type=jax.ShapeDtypeStruct((2 * B, FEAT), jnp.float32),
    mesh=_sc_mesh,
    scratch_types=[
        pltpu.VMEM((GCH, C), jnp.int32),
        pltpu.VMEM((C, FEAT), jnp.float32),
        pltpu.VMEM((C, FEAT), jnp.float32),
        pltpu.SemaphoreType.DMA,
        pltpu.SemaphoreType.DMA,
    ],
)(_gather_body)


def _node_mlp_body(p0_ref, p1_ref, w1_ref, b1_ref, w2_ref, b2_ref, out_ref):
    h = p0_ref[...] + p1_ref[...]
    h = jnp.maximum(jnp.dot(h, w1_ref[...],
                            preferred_element_type=jnp.float32) + b1_ref[...],
                    0.0)
    h = jnp.dot(h, w2_ref[...], preferred_element_type=jnp.float32) + b2_ref[...]
    out_ref[...] = jnp.maximum(h, 0.0)


def _decoder_body(e1_ref, e2_ref, wd1_ref, bd1_ref, wd2_ref, bd2_ref, out_ref):
    e1 = e1_ref[...]
    e2 = e2_ref[...]
    wa = wd1_ref[0:FEAT, :]
    wb = wd1_ref[FEAT:2 * FEAT, :]
    wc = wd1_ref[2 * FEAT:3 * FEAT, :]
    wd = wd1_ref[3 * FEAT:4 * FEAT, :]
    h = jnp.dot(e1, wa + wc, preferred_element_type=jnp.float32)
    h += jnp.dot(e2, wa + wd, preferred_element_type=jnp.float32)
    h += jnp.dot(e1 * e2, wb, preferred_element_type=jnp.float32)
    h = jnp.maximum(h + bd1_ref[...], 0.0)
    out_ref[...] = (jnp.sum(h * wd2_ref[...], axis=1, keepdims=True)
                    + bd2_ref[...])


_NODE_BLK = 1264
_DEC_BLK = 2048


def _node_mlp(p0, p1, W1, b1, W2, b2):
    return pl.pallas_call(
        _node_mlp_body,
        grid=(NPAD // _NODE_BLK,),
        in_specs=[
            pl.BlockSpec((_NODE_BLK, FEAT), lambda i: (i, 0)),
            pl.BlockSpec((_NODE_BLK, FEAT), lambda i: (i, 0)),
            pl.BlockSpec((FEAT, FEAT), lambda i: (0, 0)),
            pl.BlockSpec((1, FEAT), lambda i: (0, 0)),
            pl.BlockSpec((FEAT, FEAT), lambda i: (0, 0)),
            pl.BlockSpec((1, FEAT), lambda i: (0, 0)),
        ],
        out_specs=pl.BlockSpec((_NODE_BLK, FEAT), lambda i: (i, 0)),
        out_shape=jax.ShapeDtypeStruct((NPAD, FEAT), jnp.float32),
    )(p0, p1, W1, b1.reshape(1, FEAT), W2, b2.reshape(1, FEAT))


def _decoder(e1, e2, Wd1, bd1, Wd2, bd2):
    return pl.pallas_call(
        _decoder_body,
        grid=(B // _DEC_BLK,),
        in_specs=[
            pl.BlockSpec((_DEC_BLK, FEAT), lambda i: (i, 0)),
            pl.BlockSpec((_DEC_BLK, FEAT), lambda i: (i, 0)),
            pl.BlockSpec((4 * FEAT, FEAT), lambda i: (0, 0)),
            pl.BlockSpec((1, FEAT), lambda i: (0, 0)),
            pl.BlockSpec((1, FEAT), lambda i: (0, 0)),
            pl.BlockSpec((1, 1), lambda i: (0, 0)),
        ],
        out_specs=pl.BlockSpec((_DEC_BLK, 1), lambda i: (i, 0)),
        out_shape=jax.ShapeDtypeStruct((B, 1), jnp.float32),
    )(e1, e2, Wd1, bd1.reshape(1, FEAT), Wd2.reshape(1, FEAT),
      bd2.reshape(1, 1))


def kernel(x, edge_index, curva, idx, W1, b1, W2, b2, Wd1, bd1, Wd2, bd2):
    del curva
    pad = EPAD - E
    src = jnp.concatenate([edge_index[0], jnp.zeros((pad,), jnp.int32)])
    dst = jnp.concatenate([edge_index[1], jnp.full((pad,), NPAD, jnp.int32)])
    x_p = jnp.pad(x, ((0, NPAD - N), (0, 0)))
    ei = jnp.stack([src.reshape(NW, CH, C), dst.reshape(NW, CH, C)], axis=2)

    partials = _edge_scatter(x_p, ei)
    x1 = _node_mlp(partials[0], partials[1], W1, b1, W2, b2)

    idx_r = idx.reshape(NW, GCH, C)
    ents = _pair_gather(x1, idx_r)
    e1 = ents[:B]
    e2 = ents[B:]
    return _decoder(e1, e2, Wd1, bd1, Wd2, bd2)


# speedup vs baseline: 4.7864x; 1.0525x over previous
import functools

import jax
import jax.numpy as jnp
from jax import lax
from jax.experimental import pallas as pl
from jax.experimental.pallas import tpu as pltpu
from jax.experimental.pallas import tpu_sc as plsc

N = 10000
E = 320000
FEAT = 128
B = 16384

NC = 2
NS = 16
NW = NC * NS
C = 128

CH = -(-(E // NW) // C)
if CH % 2:
    CH += 1
EPAD = NW * CH * C

GCH = (2 * B) // (NW * C)
ROWS_PER_TILE = -(-N // (NS * 8)) * 8
NPAD = NS * ROWS_PER_TILE


def _scatter_body(x_hbm, ei_hbm, out_hbm,
                  idx_v, buf0, buf1, acc, sem0, sem1):
    c = lax.axis_index("c")
    s = lax.axis_index("s")
    w = s * NC + c
    pltpu.sync_copy(x_hbm.at[pl.ds(s * ROWS_PER_TILE, ROWS_PER_TILE)],
                    acc.at[pl.ds(s * ROWS_PER_TILE, ROWS_PER_TILE)])
    plsc.subcore_barrier()

    pltpu.sync_copy(ei_hbm.at[w, 0], idx_v.at[0])
    pltpu.async_copy(x_hbm.at[idx_v.at[0, 0]], buf0, sem0)
    pltpu.sync_copy(ei_hbm.at[w, 1], idx_v.at[1])
    pltpu.async_copy(x_hbm.at[idx_v.at[1, 0]], buf1, sem1)

    def step(i, carry):
        g = i * 2
        for b, (buf, sem) in enumerate(((buf0, sem0), (buf1, sem1))):
            j = g + b
            pltpu.make_async_copy(x_hbm.at[idx_v.at[b, 0]], buf, sem).wait()
            pltpu.sync_copy(buf, acc.at[idx_v.at[b, 1]], add=True)

            @pl.when(j + 2 < CH)
            def _():
                pltpu.sync_copy(ei_hbm.at[w, j + 2], idx_v.at[b])
                pltpu.async_copy(x_hbm.at[idx_v.at[b, 0]], buf, sem)
        return carry

    lax.fori_loop(0, CH // 2, step, 0)
    plsc.subcore_barrier()
    pltpu.sync_copy(acc.at[pl.ds(s * ROWS_PER_TILE, ROWS_PER_TILE)],
                    out_hbm.at[c, pl.ds(s * ROWS_PER_TILE, ROWS_PER_TILE)])


def _gather_body(x1_hbm, idx_hbm, out_hbm, idx_v, buf0, buf1, sem0, sem1):
    c = lax.axis_index("c")
    s = lax.axis_index("s")
    w = s * NC + c
    base = w * GCH * C
    pltpu.sync_copy(idx_hbm.at[w], idx_v)
    pltpu.async_copy(x1_hbm.at[idx_v.at[0]], buf0, sem0)
    pltpu.async_copy(x1_hbm.at[idx_v.at[1]], buf1, sem1)

    def step(i, carry):
        g = i * 2
        for b, (buf, sem) in enumerate(((buf0, sem0), (buf1, sem1))):
            j = g + b
            pltpu.make_async_copy(x1_hbm.at[idx_v.at[j]], buf, sem).wait()
            pltpu.sync_copy(buf, out_hbm.at[pl.ds(base + j * C, C)])

            @pl.when(j + 2 < GCH)
            def _():
                pltpu.async_copy(x1_hbm.at[idx_v.at[j + 2]], buf, sem)
        return carry

    lax.fori_loop(0, GCH // 2, step, 0)


_sc_mesh = plsc.VectorSubcoreMesh(core_axis_name="c", subcore_axis_name="s")

_edge_scatter = functools.partial(
    pl.kernel,
    out_type=jax.ShapeDtypeStruct((NC, NPAD, FEAT), jnp.float32),
    mesh=_sc_mesh,
    scratch_types=[
        pltpu.VMEM((2, 2, C), jnp.int32),
        pltpu.VMEM((C, FEAT), jnp.float32),
        pltpu.VMEM((C, FEAT), jnp.float32),
        pltpu.VMEM_SHARED((NPAD + 8, FEAT), jnp.float32),
        pltpu.SemaphoreType.DMA,
        pltpu.SemaphoreType.DMA,
    ],
)(_scatter_body)

_pair_gather = functools.partial(
    pl.kernel,
    out_type=jax.ShapeDtypeStruct((2 * B, FEAT), jnp.float32),
    mesh=_sc_mesh,
    scratch_types=[
        pltpu.VMEM((GCH, C), jnp.int32),
        pltpu.VMEM((C, FEAT), jnp.float32),
        pltpu.VMEM((C, FEAT), jnp.float32),
        pltpu.SemaphoreType.DMA,
        pltpu.SemaphoreType.DMA,
    ],
)(_gather_body)


def _node_mlp_body(p0_ref, p1_ref, w1_ref, b1_ref, w2_ref, b2_ref, out_ref):
    h = p0_ref[...] + p1_ref[...]
    h = jnp.maximum(jnp.dot(h, w1_ref[...],
                            preferred_element_type=jnp.float32) + b1_ref[...],
                    0.0)
    h = jnp.dot(h, w2_ref[...], preferred_element_type=jnp.float32) + b2_ref[...]
    out_ref[...] = jnp.maximum(h, 0.0)


def _decoder_body(e1_ref, e2_ref, wd1_ref, bd1_ref, wd2_ref, bd2_ref, out_ref):
    e1 = e1_ref[...]
    e2 = e2_ref[...]
    wa = wd1_ref[0:FEAT, :]
    wb = wd1_ref[FEAT:2 * FEAT, :]
    wc = wd1_ref[2 * FEAT:3 * FEAT, :]
    wd = wd1_ref[3 * FEAT:4 * FEAT, :]
    h = jnp.dot(e1, wa + wc, preferred_element_type=jnp.float32)
    h += jnp.dot(e2, wa + wd, preferred_element_type=jnp.float32)
    h += jnp.dot(e1 * e2, wb, preferred_element_type=jnp.float32)
    h = jnp.maximum(h + bd1_ref[...], 0.0)
    out_ref[...] = (jnp.sum(h * wd2_ref[...], axis=1, keepdims=True)
                    + bd2_ref[...])


_NODE_BLK = 1264
_DEC_BLK = 2048


def _node_mlp(p0, p1, W1, b1, W2, b2):
    return pl.pallas_call(
        _node_mlp_body,
        grid=(NPAD // _NODE_BLK,),
        in_specs=[
            pl.BlockSpec((_NODE_BLK, FEAT), lambda i: (i, 0)),
            pl.BlockSpec((_NODE_BLK, FEAT), lambda i: (i, 0)),
            pl.BlockSpec((FEAT, FEAT), lambda i: (0, 0)),
            pl.BlockSpec((1, FEAT), lambda i: (0, 0)),
            pl.BlockSpec((FEAT, FEAT), lambda i: (0, 0)),
            pl.BlockSpec((1, FEAT), lambda i: (0, 0)),
        ],
        out_specs=pl.BlockSpec((_NODE_BLK, FEAT), lambda i: (i, 0)),
        out_shape=jax.ShapeDtypeStruct((NPAD, FEAT), jnp.float32),
    )(p0, p1, W1, b1.reshape(1, FEAT), W2, b2.reshape(1, FEAT))


def _decoder(e1, e2, Wd1, bd1, Wd2, bd2):
    return pl.pallas_call(
        _decoder_body,
        grid=(B // _DEC_BLK,),
        in_specs=[
            pl.BlockSpec((_DEC_BLK, FEAT), lambda i: (i, 0)),
            pl.BlockSpec((_DEC_BLK, FEAT), lambda i: (i, 0)),
            pl.BlockSpec((4 * FEAT, FEAT), lambda i: (0, 0)),
            pl.BlockSpec((1, FEAT), lambda i: (0, 0)),
            pl.BlockSpec((1, FEAT), lambda i: (0, 0)),
            pl.BlockSpec((1, 1), lambda i: (0, 0)),
        ],
        out_specs=pl.BlockSpec((_DEC_BLK, 1), lambda i: (i, 0)),
        out_shape=jax.ShapeDtypeStruct((B, 1), jnp.float32),
    )(e1, e2, Wd1, bd1.reshape(1, FEAT), Wd2.reshape(1, FEAT),
      bd2.reshape(1, 1))


def kernel(x, edge_index, curva, idx, W1, b1, W2, b2, Wd1, bd1, Wd2, bd2):
    del curva
    pad = EPAD - E
    src = jnp.concatenate([edge_index[0], jnp.full((pad,), N, jnp.int32)])
    dst = jnp.concatenate([edge_index[1],
                           jnp.arange(pad, dtype=jnp.int32) % NPAD])
    x_p = jnp.pad(x, ((0, NPAD - N), (0, 0)))
    ei = jnp.stack([src.reshape(NW, CH, C), dst.reshape(NW, CH, C)], axis=2)

    partials = _edge_scatter(x_p, ei)
    x1 = _node_mlp(partials[0], partials[1], W1, b1, W2, b2)

    idx_r = idx.reshape(NW, GCH, C)
    ents = _pair_gather(x1, idx_r)
    e1 = ents[:B]
    e2 = ents[B:]
    return _decoder(e1, e2, Wd1, bd1, Wd2, bd2)


# speedup vs baseline: 11.8128x; 2.4680x over previous
import functools

import jax
import jax.numpy as jnp
from jax import lax
from jax.experimental import pallas as pl
from jax.experimental.pallas import tpu as pltpu
from jax.experimental.pallas import tpu_sc as plsc

N = 10000
E = 320000
FEAT = 128
B = 16384

NC = 2
NS = 16
NW = NC * NS
C = 128
CH = 80
PH = 2
PCH = CH // PH
EPAD = NW * CH * C

GC = 128
GCH = (2 * B) // (NW * GC)

NA = N + 8
ROWS_PER_TILE = (NA // NS // 8) * 8
REM = NA - NS * ROWS_PER_TILE


def _scatter_body(x_hbm, src_hbm, dst_hbm, out_hbm,
                  src_v, dst_v, buf0, buf1, acc, sem0, sem1):
    c = lax.axis_index("c")
    s = lax.axis_index("s")
    w = s * NC + c
    pltpu.sync_copy(x_hbm.at[pl.ds(s * ROWS_PER_TILE, ROWS_PER_TILE)],
                    acc.at[pl.ds(s * ROWS_PER_TILE, ROWS_PER_TILE)])

    @pl.when(s == NS - 1)
    def _():
        pltpu.sync_copy(x_hbm.at[pl.ds(NS * ROWS_PER_TILE, REM)],
                        acc.at[pl.ds(NS * ROWS_PER_TILE, REM)])

    plsc.subcore_barrier()

    for p in range(PH):
        pltpu.sync_copy(src_hbm.at[w, pl.ds(p * PCH, PCH)], src_v)
        pltpu.sync_copy(dst_hbm.at[w, pl.ds(p * PCH, PCH)], dst_v)
        pltpu.async_copy(x_hbm.at[src_v.at[0]], buf0, sem0)
        pltpu.async_copy(x_hbm.at[src_v.at[1]], buf1, sem1)

        def step(i, carry):
            g = i * 2
            for b, (buf, sem) in enumerate(((buf0, sem0), (buf1, sem1))):
                j = g + b
                pltpu.make_async_copy(x_hbm.at[src_v.at[j]], buf, sem).wait()
                pltpu.sync_copy(buf, acc.at[dst_v.at[j]], add=True)
                pltpu.async_copy(x_hbm.at[src_v.at[j + 2]], buf, sem)
            return carry

        lax.fori_loop(0, PCH // 2 - 1, step, 0)
        for b, (buf, sem) in enumerate(((buf0, sem0), (buf1, sem1))):
            j = PCH - 2 + b
            pltpu.make_async_copy(x_hbm.at[src_v.at[j]], buf, sem).wait()
            pltpu.sync_copy(buf, acc.at[dst_v.at[j]], add=True)

    plsc.subcore_barrier()
    pltpu.sync_copy(acc.at[pl.ds(s * ROWS_PER_TILE, ROWS_PER_TILE)],
                    out_hbm.at[c, pl.ds(s * ROWS_PER_TILE, ROWS_PER_TILE)])

    @pl.when(s == NS - 1)
    def _():
        pltpu.sync_copy(acc.at[pl.ds(NS * ROWS_PER_TILE, REM)],
                        out_hbm.at[c, pl.ds(NS * ROWS_PER_TILE, REM)])


def _gather_body(x1_hbm, idx_hbm, out_hbm, idx_v, buf0, buf1, sem0, sem1):
    c = lax.axis_index("c")
    s = lax.axis_index("s")
    w = s * NC + c
    base = w * GCH * GC
    pltpu.sync_copy(idx_hbm.at[w], idx_v)
    pltpu.async_copy(x1_hbm.at[idx_v.at[0]], buf0, sem0)
    pltpu.async_copy(x1_hbm.at[idx_v.at[1]], buf1, sem1)

    def step(i, carry):
        g = i * 2
        for b, (buf, sem) in enumerate(((buf0, sem0), (buf1, sem1))):
            j = g + b
            pltpu.make_async_copy(x1_hbm.at[idx_v.at[j]], buf, sem).wait()
            pltpu.sync_copy(buf, out_hbm.at[pl.ds(base + j * GC, GC)])

            @pl.when(j + 2 < GCH)
            def _():
                pltpu.async_copy(x1_hbm.at[idx_v.at[j + 2]], buf, sem)
        return carry

    lax.fori_loop(0, GCH // 2, step, 0)


_sc_mesh = plsc.VectorSubcoreMesh(core_axis_name="c", subcore_axis_name="s")

_edge_scatter = functools.partial(
    pl.kernel,
    out_type=jax.ShapeDtypeStruct((NC, NA, FEAT), jnp.float32),
    mesh=_sc_mesh,
    scratch_types=[
        pltpu.VMEM((PCH, C), jnp.int32),
        pltpu.VMEM((PCH, C), jnp.int32),
        pltpu.VMEM((C, FEAT), jnp.float32),
        pltpu.VMEM((C, FEAT), jnp.float32),
        pltpu.VMEM_SHARED((NA, FEAT), jnp.float32),
        pltpu.SemaphoreType.DMA,
        pltpu.SemaphoreType.DMA,
    ],
)(_scatter_body)

_pair_gather = functools.partial(
    pl.kernel,
    out_type=jax.ShapeDtypeStruct((2 * B, FEAT), jnp.float32),
    mesh=_sc_mesh,
    scratch_types=[
        pltpu.VMEM((GCH, GC), jnp.int32),
        pltpu.VMEM((GC, FEAT), jnp.float32),
        pltpu.VMEM((GC, FEAT), jnp.float32),
        pltpu.SemaphoreType.DMA,
        pltpu.SemaphoreType.DMA,
    ],
)(_gather_body)


def _node_mlp_body(p0_ref, p1_ref, w1_ref, b1_ref, w2_ref, b2_ref, out_ref):
    h = p0_ref[...] + p1_ref[...]
    h = jnp.maximum(jnp.dot(h, w1_ref[...],
                            preferred_element_type=jnp.float32) + b1_ref[...],
                    0.0)
    h = jnp.dot(h, w2_ref[...], preferred_element_type=jnp.float32) + b2_ref[...]
    out_ref[...] = jnp.maximum(h, 0.0)


def _decoder_body(e1_ref, e2_ref, wd1_ref, bd1_ref, wd2_ref, bd2_ref, out_ref):
    e1 = e1_ref[...]
    e2 = e2_ref[...]
    wa = wd1_ref[0:FEAT, :]
    wb = wd1_ref[FEAT:2 * FEAT, :]
    wc = wd1_ref[2 * FEAT:3 * FEAT, :]
    wd = wd1_ref[3 * FEAT:4 * FEAT, :]
    h = jnp.dot(e1, wa + wc, preferred_element_type=jnp.float32)
    h += jnp.dot(e2, wa + wd, preferred_element_type=jnp.float32)
    h += jnp.dot(e1 * e2, wb, preferred_element_type=jnp.float32)
    h = jnp.maximum(h + bd1_ref[...], 0.0)
    out_ref[...] = (jnp.sum(h * wd2_ref[...], axis=1, keepdims=True)
                    + bd2_ref[...])


_NODE_BLK = 1112
_DEC_BLK = 2048


def _node_mlp(p0, p1, W1, b1, W2, b2):
    return pl.pallas_call(
        _node_mlp_body,
        grid=(NA // _NODE_BLK,),
        in_specs=[
            pl.BlockSpec((_NODE_BLK, FEAT), lambda i: (i, 0)),
            pl.BlockSpec((_NODE_BLK, FEAT), lambda i: (i, 0)),
            pl.BlockSpec((FEAT, FEAT), lambda i: (0, 0)),
            pl.BlockSpec((1, FEAT), lambda i: (0, 0)),
            pl.BlockSpec((FEAT, FEAT), lambda i: (0, 0)),
            pl.BlockSpec((1, FEAT), lambda i: (0, 0)),
        ],
        out_specs=pl.BlockSpec((_NODE_BLK, FEAT), lambda i: (i, 0)),
        out_shape=jax.ShapeDtypeStruct((NA, FEAT), jnp.float32),
    )(p0, p1, W1, b1.reshape(1, FEAT), W2, b2.reshape(1, FEAT))


def _decoder(e1, e2, Wd1, bd1, Wd2, bd2):
    return pl.pallas_call(
        _decoder_body,
        grid=(B // _DEC_BLK,),
        in_specs=[
            pl.BlockSpec((_DEC_BLK, FEAT), lambda i: (i, 0)),
            pl.BlockSpec((_DEC_BLK, FEAT), lambda i: (i, 0)),
            pl.BlockSpec((4 * FEAT, FEAT), lambda i: (0, 0)),
            pl.BlockSpec((1, FEAT), lambda i: (0, 0)),
            pl.BlockSpec((1, FEAT), lambda i: (0, 0)),
            pl.BlockSpec((1, 1), lambda i: (0, 0)),
        ],
        out_specs=pl.BlockSpec((_DEC_BLK, 1), lambda i: (i, 0)),
        out_shape=jax.ShapeDtypeStruct((B, 1), jnp.float32),
    )(e1, e2, Wd1, bd1.reshape(1, FEAT), Wd2.reshape(1, FEAT),
      bd2.reshape(1, 1))


def kernel(x, edge_index, curva, idx, W1, b1, W2, b2, Wd1, bd1, Wd2, bd2):
    del curva
    padn = EPAD - E
    ar = jnp.arange(padn, dtype=jnp.int32)
    src = jnp.concatenate([edge_index[0], N + (ar % 8)])
    dst = jnp.concatenate([edge_index[1], ar % N])
    src_r = src.reshape(NW, CH, C)
    dst_r = dst.reshape(NW, CH, C)
    x_p = jnp.pad(x, ((0, NA - N), (0, 0)))

    partials = _edge_scatter(x_p, src_r, dst_r)
    x1 = _node_mlp(partials[0], partials[1], W1, b1, W2, b2)

    idx_r = idx.reshape(NW, GCH, GC)
    ents = _pair_gather(x1, idx_r)
    e1 = ents[:B]
    e2 = ents[B:]
    return _decoder(e1, e2, Wd1, bd1, Wd2, bd2)


# speedup vs baseline: 13.3109x; 1.1268x over previous
import functools

import numpy as np

import jax
import jax.numpy as jnp
from jax import lax
from jax.experimental import pallas as pl
from jax.experimental.pallas import tpu as pltpu
from jax.experimental.pallas import tpu_sc as plsc

N = 10000
E = 320000
FEAT = 128
B = 16384

NC = 2
NS = 16
NW = NC * NS
C = 128
CH = 80
PH = 2
PCH = CH // PH
EPAD = NW * CH * C

GC = 128
GCH = (2 * B) // (NW * GC)

NA = N + 8
ROWS_PER_TILE = (NA // NS // 8) * 8
REM = NA - NS * ROWS_PER_TILE


def _scatter_body(x_hbm, ei_hbm, out_hbm,
                  src_v, dst_v, buf0, buf1, acc, sem0, sem1):
    c = lax.axis_index("c")
    s = lax.axis_index("s")
    w = s * NC + c
    pltpu.sync_copy(x_hbm.at[pl.ds(s * ROWS_PER_TILE, ROWS_PER_TILE)],
                    acc.at[pl.ds(s * ROWS_PER_TILE, ROWS_PER_TILE)])

    @pl.when(s == NS - 1)
    def _():
        pltpu.sync_copy(x_hbm.at[pl.ds(NS * ROWS_PER_TILE, REM)],
                        acc.at[pl.ds(NS * ROWS_PER_TILE, REM)])

    plsc.subcore_barrier()

    for p in range(PH):
        pltpu.sync_copy(ei_hbm.at[0, w, pl.ds(p * PCH, PCH)], src_v)
        pltpu.sync_copy(ei_hbm.at[1, w, pl.ds(p * PCH, PCH)], dst_v)
        pltpu.async_copy(x_hbm.at[src_v.at[0]], buf0, sem0)
        pltpu.async_copy(x_hbm.at[src_v.at[1]], buf1, sem1)

        def step(i, carry):
            g = i * 2
            for b, (buf, sem) in enumerate(((buf0, sem0), (buf1, sem1))):
                j = g + b
                pltpu.make_async_copy(x_hbm.at[src_v.at[j]], buf, sem).wait()
                pltpu.sync_copy(buf, acc.at[dst_v.at[j]], add=True)
                pltpu.async_copy(x_hbm.at[src_v.at[j + 2]], buf, sem)
            return carry

        lax.fori_loop(0, PCH // 2 - 1, step, 0)
        for b, (buf, sem) in enumerate(((buf0, sem0), (buf1, sem1))):
            j = PCH - 2 + b
            pltpu.make_async_copy(x_hbm.at[src_v.at[j]], buf, sem).wait()
            pltpu.sync_copy(buf, acc.at[dst_v.at[j]], add=True)

    plsc.subcore_barrier()
    pltpu.sync_copy(acc.at[pl.ds(s * ROWS_PER_TILE, ROWS_PER_TILE)],
                    out_hbm.at[c, pl.ds(s * ROWS_PER_TILE, ROWS_PER_TILE)])

    @pl.when(s == NS - 1)
    def _():
        pltpu.sync_copy(acc.at[pl.ds(NS * ROWS_PER_TILE, REM)],
                        out_hbm.at[c, pl.ds(NS * ROWS_PER_TILE, REM)])


def _gather_body(x1_hbm, idx_hbm, out_hbm, idx_v, buf0, buf1, sem0, sem1):
    c = lax.axis_index("c")
    s = lax.axis_index("s")
    w = s * NC + c
    base = w * GCH * GC
    pltpu.sync_copy(idx_hbm.at[w], idx_v)
    pltpu.async_copy(x1_hbm.at[idx_v.at[0]], buf0, sem0)
    pltpu.async_copy(x1_hbm.at[idx_v.at[1]], buf1, sem1)

    def step(i, carry):
        g = i * 2
        for b, (buf, sem) in enumerate(((buf0, sem0), (buf1, sem1))):
            j = g + b
            pltpu.make_async_copy(x1_hbm.at[idx_v.at[j]], buf, sem).wait()
            pltpu.sync_copy(buf, out_hbm.at[pl.ds(base + j * GC, GC)])

            @pl.when(j + 2 < GCH)
            def _():
                pltpu.async_copy(x1_hbm.at[idx_v.at[j + 2]], buf, sem)
        return carry

    lax.fori_loop(0, GCH // 2, step, 0)


_sc_mesh = plsc.VectorSubcoreMesh(core_axis_name="c", subcore_axis_name="s")

_edge_scatter = functools.partial(
    pl.kernel,
    out_type=jax.ShapeDtypeStruct((NC, NA, FEAT), jnp.float32),
    mesh=_sc_mesh,
    scratch_types=[
        pltpu.VMEM((PCH, C), jnp.int32),
        pltpu.VMEM((PCH, C), jnp.int32),
        pltpu.VMEM((C, FEAT), jnp.float32),
        pltpu.VMEM((C, FEAT), jnp.float32),
        pltpu.VMEM_SHARED((NA, FEAT), jnp.float32),
        pltpu.SemaphoreType.DMA,
        pltpu.SemaphoreType.DMA,
    ],
)(_scatter_body)

_pair_gather = functools.partial(
    pl.kernel,
    out_type=jax.ShapeDtypeStruct((2 * B, FEAT), jnp.float32),
    mesh=_sc_mesh,
    scratch_types=[
        pltpu.VMEM((GCH, GC), jnp.int32),
        pltpu.VMEM((GC, FEAT), jnp.float32),
        pltpu.VMEM((GC, FEAT), jnp.float32),
        pltpu.SemaphoreType.DMA,
        pltpu.SemaphoreType.DMA,
    ],
)(_gather_body)


def _node_mlp_body(p_ref, w1_ref, b1_ref, w2_ref, b2_ref, out_ref):
    h = p_ref[0] + p_ref[1]
    h = jnp.maximum(jnp.dot(h, w1_ref[...],
                            preferred_element_type=jnp.float32) + b1_ref[...],
                    0.0)
    h = jnp.dot(h, w2_ref[...], preferred_element_type=jnp.float32) + b2_ref[...]
    out_ref[...] = jnp.maximum(h, 0.0)


def _decoder_body(ee_ref, wd1_ref, bd1_ref, wd2_ref, bd2_ref, out_ref):
    e1 = ee_ref[0]
    e2 = ee_ref[1]
    wa = wd1_ref[0:FEAT, :]
    wb = wd1_ref[FEAT:2 * FEAT, :]
    wc = wd1_ref[2 * FEAT:3 * FEAT, :]
    wd = wd1_ref[3 * FEAT:4 * FEAT, :]
    h = jnp.dot(e1, wa + wc, preferred_element_type=jnp.float32)
    h += jnp.dot(e2, wa + wd, preferred_element_type=jnp.float32)
    h += jnp.dot(e1 * e2, wb, preferred_element_type=jnp.float32)
    h = jnp.maximum(h + bd1_ref[...], 0.0)
    out_ref[...] = (jnp.sum(h * wd2_ref[...], axis=1, keepdims=True)
                    + bd2_ref[...])


_NODE_BLK = 1112
_DEC_BLK = 2048


def _node_mlp(partials, W1, b1, W2, b2):
    return pl.pallas_call(
        _node_mlp_body,
        grid=(NA // _NODE_BLK,),
        in_specs=[
            pl.BlockSpec((2, _NODE_BLK, FEAT), lambda i: (0, i, 0)),
            pl.BlockSpec((FEAT, FEAT), lambda i: (0, 0)),
            pl.BlockSpec((1, FEAT), lambda i: (0, 0)),
            pl.BlockSpec((FEAT, FEAT), lambda i: (0, 0)),
            pl.BlockSpec((1, FEAT), lambda i: (0, 0)),
        ],
        out_specs=pl.BlockSpec((_NODE_BLK, FEAT), lambda i: (i, 0)),
        out_shape=jax.ShapeDtypeStruct((NA, FEAT), jnp.float32),
    )(partials, W1, b1.reshape(1, FEAT), W2, b2.reshape(1, FEAT))


def _decoder(ee, Wd1, bd1, Wd2, bd2):
    return pl.pallas_call(
        _decoder_body,
        grid=(B // _DEC_BLK,),
        in_specs=[
            pl.BlockSpec((2, _DEC_BLK, FEAT), lambda i: (0, i, 0)),
            pl.BlockSpec((4 * FEAT, FEAT), lambda i: (0, 0)),
            pl.BlockSpec((1, FEAT), lambda i: (0, 0)),
            pl.BlockSpec((1, FEAT), lambda i: (0, 0)),
            pl.BlockSpec((1, 1), lambda i: (0, 0)),
        ],
        out_specs=pl.BlockSpec((_DEC_BLK, 1), lambda i: (i, 0)),
        out_shape=jax.ShapeDtypeStruct((B, 1), jnp.float32),
    )(ee, Wd1, bd1.reshape(1, FEAT), Wd2.reshape(1, FEAT),
      bd2.reshape(1, 1))


_AR = np.arange(EPAD - E, dtype=np.int32)
_PAD_EDGES = np.stack([N + (_AR % (NA - N)), _AR % N]).astype(np.int32)


def kernel(x, edge_index, curva, idx, W1, b1, W2, b2, Wd1, bd1, Wd2, bd2):
    del curva
    ei = jnp.concatenate([edge_index, jnp.asarray(_PAD_EDGES)],
                         axis=1).reshape(2, NW, CH, C)
    x_p = jnp.pad(x, ((0, NA - N), (0, 0)))

    partials = _edge_scatter(x_p, ei)
    x1 = _node_mlp(partials, W1, b1, W2, b2)

    idx_r = idx.reshape(NW, GCH, GC)
    ents = _pair_gather(x1, idx_r)
    return _decoder(ents.reshape(2, B, FEAT), Wd1, bd1, Wd2, bd2)


# speedup vs baseline: 13.5436x; 1.0175x over previous
import functools

import numpy as np

import jax
import jax.numpy as jnp
from jax import lax
from jax.experimental import pallas as pl
from jax.experimental.pallas import tpu as pltpu
from jax.experimental.pallas import tpu_sc as plsc

N = 10000
E = 320000
FEAT = 128
B = 16384

NC = 2
NS = 16
NW = NC * NS
C = 128
CH = 80
SUB = 8
NPH = CH // SUB
EPAD = NW * CH * C

GC = 128
GCH = (2 * B) // (NW * GC)

NA = N + 8
ROWS_PER_TILE = (NA // NS // 8) * 8
REM = NA - NS * ROWS_PER_TILE


def _scatter_body(x_hbm, ei_hbm, out_hbm,
                  idx_v, buf0, buf1, acc, sem0, sem1, semi):
    c = lax.axis_index("c")
    s = lax.axis_index("s")
    w = s * NC + c

    def stage_idx(phase, slot):
        return (
            pltpu.async_copy(ei_hbm.at[0, w, pl.ds(phase * SUB, SUB)],
                             idx_v.at[slot, 0], semi),
            pltpu.async_copy(ei_hbm.at[1, w, pl.ds(phase * SUB, SUB)],
                             idx_v.at[slot, 1], semi),
        )

    def wait_idx(phase, slot):
        pltpu.make_async_copy(ei_hbm.at[0, w, pl.ds(phase * SUB, SUB)],
                              idx_v.at[slot, 0], semi).wait()
        pltpu.make_async_copy(ei_hbm.at[1, w, pl.ds(phase * SUB, SUB)],
                              idx_v.at[slot, 1], semi).wait()

    stage_idx(0, 0)
    stage_idx(1, 1)
    pltpu.sync_copy(x_hbm.at[pl.ds(s * ROWS_PER_TILE, ROWS_PER_TILE)],
                    acc.at[pl.ds(s * ROWS_PER_TILE, ROWS_PER_TILE)])

    @pl.when(s == NS - 1)
    def _():
        pltpu.sync_copy(x_hbm.at[pl.ds(NS * ROWS_PER_TILE, REM)],
                        acc.at[pl.ds(NS * ROWS_PER_TILE, REM)])

    plsc.subcore_barrier()

    wait_idx(0, 0)
    pltpu.async_copy(x_hbm.at[idx_v.at[0, 0, 0]], buf0, sem0)
    pltpu.async_copy(x_hbm.at[idx_v.at[0, 0, 1]], buf1, sem1)

    for p in range(NPH):
        sl = p % 2
        nsl = (p + 1) % 2

        def step(i, carry, sl=sl):
            for b, (buf, sem) in enumerate(((buf0, sem0), (buf1, sem1))):
                k = i * 2 + b
                pltpu.make_async_copy(x_hbm.at[idx_v.at[sl, 0, k]],
                                      buf, sem).wait()
                pltpu.sync_copy(buf, acc.at[idx_v.at[sl, 1, k]], add=True)
                pltpu.async_copy(x_hbm.at[idx_v.at[sl, 0, k + 2]], buf, sem)
            return carry

        lax.fori_loop(0, SUB // 2 - 1, step, 0)
        if p + 1 < NPH:
            wait_idx(p + 1, nsl)
        for b, (buf, sem) in enumerate(((buf0, sem0), (buf1, sem1))):
            k = SUB - 2 + b
            pltpu.make_async_copy(x_hbm.at[idx_v.at[sl, 0, k]],
                                  buf, sem).wait()
            pltpu.sync_copy(buf, acc.at[idx_v.at[sl, 1, k]], add=True)
            if p + 1 < NPH:
                pltpu.async_copy(x_hbm.at[idx_v.at[nsl, 0, b]], buf, sem)
        if p + 2 < NPH:
            stage_idx(p + 2, sl)

    plsc.subcore_barrier()
    pltpu.sync_copy(acc.at[pl.ds(s * ROWS_PER_TILE, ROWS_PER_TILE)],
                    out_hbm.at[c, pl.ds(s * ROWS_PER_TILE, ROWS_PER_TILE)])

    @pl.when(s == NS - 1)
    def _():
        pltpu.sync_copy(acc.at[pl.ds(NS * ROWS_PER_TILE, REM)],
                        out_hbm.at[c, pl.ds(NS * ROWS_PER_TILE, REM)])


def _gather_body(x1_hbm, idx_hbm, out_hbm, idx_v, buf0, buf1, sem0, sem1):
    c = lax.axis_index("c")
    s = lax.axis_index("s")
    w = s * NC + c
    base = w * GCH * GC
    pltpu.sync_copy(idx_hbm.at[w], idx_v)
    pltpu.async_copy(x1_hbm.at[idx_v.at[0]], buf0, sem0)
    pltpu.async_copy(x1_hbm.at[idx_v.at[1]], buf1, sem1)

    def step(i, carry):
        g = i * 2
        for b, (buf, sem) in enumerate(((buf0, sem0), (buf1, sem1))):
            j = g + b
            pltpu.make_async_copy(x1_hbm.at[idx_v.at[j]], buf, sem).wait()
            pltpu.sync_copy(buf, out_hbm.at[pl.ds(base + j * GC, GC)])

            @pl.when(j + 2 < GCH)
            def _():
                pltpu.async_copy(x1_hbm.at[idx_v.at[j + 2]], buf, sem)
        return carry

    lax.fori_loop(0, GCH // 2, step, 0)


_sc_mesh = plsc.VectorSubcoreMesh(core_axis_name="c", subcore_axis_name="s")

_edge_scatter = functools.partial(
    pl.kernel,
    out_type=jax.ShapeDtypeStruct((NC, NA, FEAT), jnp.float32),
    mesh=_sc_mesh,
    scratch_types=[
        pltpu.VMEM((2, 2, SUB, C), jnp.int32),
        pltpu.VMEM((C, FEAT), jnp.float32),
        pltpu.VMEM((C, FEAT), jnp.float32),
        pltpu.VMEM_SHARED((NA, FEAT), jnp.float32),
        pltpu.SemaphoreType.DMA,
        pltpu.SemaphoreType.DMA,
        pltpu.SemaphoreType.DMA,
    ],
)(_scatter_body)

_pair_gather = functools.partial(
    pl.kernel,
    out_type=jax.ShapeDtypeStruct((2 * B, FEAT), jnp.float32),
    mesh=_sc_mesh,
    scratch_types=[
        pltpu.VMEM((GCH, GC), jnp.int32),
        pltpu.VMEM((GC, FEAT), jnp.float32),
        pltpu.VMEM((GC, FEAT), jnp.float32),
        pltpu.SemaphoreType.DMA,
        pltpu.SemaphoreType.DMA,
    ],
)(_gather_body)


def _node_mlp_body(p_ref, w1_ref, b1_ref, w2_ref, b2_ref, out_ref):
    h = p_ref[0] + p_ref[1]
    h = jnp.maximum(jnp.dot(h, w1_ref[...],
                            preferred_element_type=jnp.float32) + b1_ref[...],
                    0.0)
    h = jnp.dot(h, w2_ref[...], preferred_element_type=jnp.float32) + b2_ref[...]
    out_ref[...] = jnp.maximum(h, 0.0)


def _decoder_body(ee_ref, wd1_ref, bd1_ref, wd2_ref, bd2_ref, out_ref):
    e1 = ee_ref[0]
    e2 = ee_ref[1]
    wa = wd1_ref[0:FEAT, :]
    wb = wd1_ref[FEAT:2 * FEAT, :]
    wc = wd1_ref[2 * FEAT:3 * FEAT, :]
    wd = wd1_ref[3 * FEAT:4 * FEAT, :]
    h = jnp.dot(e1, wa + wc, preferred_element_type=jnp.float32)
    h += jnp.dot(e2, wa + wd, preferred_element_type=jnp.float32)
    h += jnp.dot(e1 * e2, wb, preferred_element_type=jnp.float32)
    h = jnp.maximum(h + bd1_ref[...], 0.0)
    out_ref[...] = (jnp.sum(h * wd2_ref[...], axis=1, keepdims=True)
                    + bd2_ref[...])


_NODE_BLK = 1112
_DEC_BLK = 2048


def _node_mlp(partials, W1, b1, W2, b2):
    return pl.pallas_call(
        _node_mlp_body,
        grid=(NA // _NODE_BLK,),
        in_specs=[
            pl.BlockSpec((2, _NODE_BLK, FEAT), lambda i: (0, i, 0)),
            pl.BlockSpec((FEAT, FEAT), lambda i: (0, 0)),
            pl.BlockSpec((1, FEAT), lambda i: (0, 0)),
            pl.BlockSpec((FEAT, FEAT), lambda i: (0, 0)),
            pl.BlockSpec((1, FEAT), lambda i: (0, 0)),
        ],
        out_specs=pl.BlockSpec((_NODE_BLK, FEAT), lambda i: (i, 0)),
        out_shape=jax.ShapeDtypeStruct((NA, FEAT), jnp.float32),
    )(partials, W1, b1.reshape(1, FEAT), W2, b2.reshape(1, FEAT))


def _decoder(ee, Wd1, bd1, Wd2, bd2):
    return pl.pallas_call(
        _decoder_body,
        grid=(B // _DEC_BLK,),
        in_specs=[
            pl.BlockSpec((2, _DEC_BLK, FEAT), lambda i: (0, i, 0)),
            pl.BlockSpec((4 * FEAT, FEAT), lambda i: (0, 0)),
            pl.BlockSpec((1, FEAT), lambda i: (0, 0)),
            pl.BlockSpec((1, FEAT), lambda i: (0, 0)),
            pl.BlockSpec((1, 1), lambda i: (0, 0)),
        ],
        out_specs=pl.BlockSpec((_DEC_BLK, 1), lambda i: (i, 0)),
        out_shape=jax.ShapeDtypeStruct((B, 1), jnp.float32),
    )(ee, Wd1, bd1.reshape(1, FEAT), Wd2.reshape(1, FEAT),
      bd2.reshape(1, 1))


_AR = np.arange(EPAD - E, dtype=np.int32)
_PAD_EDGES = np.stack([N + (_AR % (NA - N)), _AR % N]).astype(np.int32)


def kernel(x, edge_index, curva, idx, W1, b1, W2, b2, Wd1, bd1, Wd2, bd2):
    del curva
    ei = jnp.concatenate([edge_index, jnp.asarray(_PAD_EDGES)],
                         axis=1).reshape(2, NW, CH, C)
    x_p = jnp.pad(x, ((0, NA - N), (0, 0)))

    partials = _edge_scatter(x_p, ei)
    x1 = _node_mlp(partials, W1, b1, W2, b2)

    idx_r = idx.reshape(NW, GCH, GC)
    ents = _pair_gather(x1, idx_r)
    return _decoder(ents.reshape(2, B, FEAT), Wd1, bd1, Wd2, bd2)


# speedup vs baseline: 13.5650x; 1.0016x over previous
import functools

import numpy as np

import jax
import jax.numpy as jnp
from jax import lax
from jax.experimental import pallas as pl
from jax.experimental.pallas import tpu as pltpu
from jax.experimental.pallas import tpu_sc as plsc

N = 10000
E = 320000
FEAT = 128
B = 16384

NC = 2
NS = 16
NW = NC * NS
C = 128
CH = 80
SUB = 8
NPH = CH // SUB
EPAD = NW * CH * C

GC = 128
GCH = (2 * B) // (NW * GC)

NA = N + 8
ROWS_PER_TILE = (NA // NS // 8) * 8
REM = NA - NS * ROWS_PER_TILE


def _scatter_body(x_hbm, ei_hbm, out_hbm,
                  idx_v, buf0, buf1, acc, sem0, sem1, semi):
    c = lax.axis_index("c")
    s = lax.axis_index("s")
    w = s * NC + c

    def stage_idx(phase, slot):
        return (
            pltpu.async_copy(ei_hbm.at[0, w, pl.ds(phase * SUB, SUB)],
                             idx_v.at[slot, 0], semi),
            pltpu.async_copy(ei_hbm.at[1, w, pl.ds(phase * SUB, SUB)],
                             idx_v.at[slot, 1], semi),
        )

    def wait_idx(phase, slot):
        pltpu.make_async_copy(ei_hbm.at[0, w, pl.ds(phase * SUB, SUB)],
                              idx_v.at[slot, 0], semi).wait()
        pltpu.make_async_copy(ei_hbm.at[1, w, pl.ds(phase * SUB, SUB)],
                              idx_v.at[slot, 1], semi).wait()

    stage_idx(0, 0)
    stage_idx(1, 1)
    pltpu.sync_copy(x_hbm.at[pl.ds(s * ROWS_PER_TILE, ROWS_PER_TILE)],
                    acc.at[pl.ds(s * ROWS_PER_TILE, ROWS_PER_TILE)])

    @pl.when(s == NS - 1)
    def _():
        pltpu.sync_copy(x_hbm.at[pl.ds(NS * ROWS_PER_TILE, REM)],
                        acc.at[pl.ds(NS * ROWS_PER_TILE, REM)])

    plsc.subcore_barrier()

    wait_idx(0, 0)
    pltpu.async_copy(x_hbm.at[idx_v.at[0, 0, 0]], buf0, sem0)
    pltpu.async_copy(x_hbm.at[idx_v.at[0, 0, 1]], buf1, sem1)

    for p in range(NPH):
        sl = p % 2
        nsl = (p + 1) % 2

        def step(i, carry, sl=sl):
            for b, (buf, sem) in enumerate(((buf0, sem0), (buf1, sem1))):
                k = i * 2 + b
                pltpu.make_async_copy(x_hbm.at[idx_v.at[sl, 0, k]],
                                      buf, sem).wait()
                pltpu.sync_copy(buf, acc.at[idx_v.at[sl, 1, k]], add=True)
                pltpu.async_copy(x_hbm.at[idx_v.at[sl, 0, k + 2]], buf, sem)
            return carry

        lax.fori_loop(0, SUB // 2 - 1, step, 0)
        if p + 1 < NPH:
            wait_idx(p + 1, nsl)
        for b, (buf, sem) in enumerate(((buf0, sem0), (buf1, sem1))):
            k = SUB - 2 + b
            pltpu.make_async_copy(x_hbm.at[idx_v.at[sl, 0, k]],
                                  buf, sem).wait()
            pltpu.sync_copy(buf, acc.at[idx_v.at[sl, 1, k]], add=True)
            if p + 1 < NPH:
                pltpu.async_copy(x_hbm.at[idx_v.at[nsl, 0, b]], buf, sem)
        if p + 2 < NPH:
            stage_idx(p + 2, sl)

    plsc.subcore_barrier()
    pltpu.sync_copy(acc.at[pl.ds(s * ROWS_PER_TILE, ROWS_PER_TILE)],
                    out_hbm.at[c, pl.ds(s * ROWS_PER_TILE, ROWS_PER_TILE)])

    @pl.when(s == NS - 1)
    def _():
        pltpu.sync_copy(acc.at[pl.ds(NS * ROWS_PER_TILE, REM)],
                        out_hbm.at[c, pl.ds(NS * ROWS_PER_TILE, REM)])


def _gather_body(x1_hbm, idx_hbm, out_hbm, idx_v, buf0, buf1, sem0, sem1):
    c = lax.axis_index("c")
    s = lax.axis_index("s")
    w = s * NC + c
    base = w * GCH * GC
    pltpu.sync_copy(idx_hbm.at[w], idx_v)
    pltpu.async_copy(x1_hbm.at[idx_v.at[0]], buf0, sem0)
    pltpu.async_copy(x1_hbm.at[idx_v.at[1]], buf1, sem1)

    def step(i, carry):
        g = i * 2
        for b, (buf, sem) in enumerate(((buf0, sem0), (buf1, sem1))):
            j = g + b
            pltpu.make_async_copy(x1_hbm.at[idx_v.at[j]], buf, sem).wait()
            pltpu.sync_copy(buf, out_hbm.at[pl.ds(base + j * GC, GC)])

            @pl.when(j + 2 < GCH)
            def _():
                pltpu.async_copy(x1_hbm.at[idx_v.at[j + 2]], buf, sem)
        return carry

    lax.fori_loop(0, GCH // 2, step, 0)


_sc_mesh = plsc.VectorSubcoreMesh(core_axis_name="c", subcore_axis_name="s")

_edge_scatter = functools.partial(
    pl.kernel,
    out_type=jax.ShapeDtypeStruct((NC, NA, FEAT), jnp.float32),
    mesh=_sc_mesh,
    scratch_types=[
        pltpu.VMEM((2, 2, SUB, C), jnp.int32),
        pltpu.VMEM((C, FEAT), jnp.float32),
        pltpu.VMEM((C, FEAT), jnp.float32),
        pltpu.VMEM_SHARED((NA, FEAT), jnp.float32),
        pltpu.SemaphoreType.DMA,
        pltpu.SemaphoreType.DMA,
        pltpu.SemaphoreType.DMA,
    ],
)(_scatter_body)

_pair_gather = functools.partial(
    pl.kernel,
    out_type=jax.ShapeDtypeStruct((2 * B, FEAT), jnp.float32),
    mesh=_sc_mesh,
    scratch_types=[
        pltpu.VMEM((GCH, GC), jnp.int32),
        pltpu.VMEM((GC, FEAT), jnp.float32),
        pltpu.VMEM((GC, FEAT), jnp.float32),
        pltpu.SemaphoreType.DMA,
        pltpu.SemaphoreType.DMA,
    ],
)(_gather_body)


def _node_mlp_body(p_ref, w1_ref, b1_ref, w2_ref, b2_ref, out_ref):
    h = p_ref[0] + p_ref[1]
    h = jnp.maximum(jnp.dot(h, w1_ref[...],
                            preferred_element_type=jnp.float32) + b1_ref[...],
                    0.0)
    h = jnp.dot(h, w2_ref[...], preferred_element_type=jnp.float32) + b2_ref[...]
    out_ref[...] = jnp.maximum(h, 0.0)


def _decoder_body(ee_ref, wd1_ref, bd1_ref, wd2_ref, bd2_ref, out_ref):
    e1 = ee_ref[0]
    e2 = ee_ref[1]
    e1b = e1.astype(jnp.bfloat16)
    e2b = e2.astype(jnp.bfloat16)
    p12 = (e1 * e2).astype(jnp.bfloat16)
    wa = wd1_ref[0:FEAT, :]
    wb = wd1_ref[FEAT:2 * FEAT, :]
    wc = wd1_ref[2 * FEAT:3 * FEAT, :]
    wd = wd1_ref[3 * FEAT:4 * FEAT, :]
    h = jnp.dot(e1b, (wa + wc).astype(jnp.bfloat16),
                preferred_element_type=jnp.float32)
    h += jnp.dot(e2b, (wa + wd).astype(jnp.bfloat16),
                 preferred_element_type=jnp.float32)
    h += jnp.dot(p12, wb.astype(jnp.bfloat16),
                 preferred_element_type=jnp.float32)
    h = jnp.maximum(h + bd1_ref[...], 0.0)
    out_ref[...] = (jnp.sum(h * wd2_ref[...], axis=1, keepdims=True)
                    + bd2_ref[...])


_NODE_BLK = 1112
_DEC_BLK = 2048


def _node_mlp(partials, W1, b1, W2, b2):
    return pl.pallas_call(
        _node_mlp_body,
        grid=(NA // _NODE_BLK,),
        in_specs=[
            pl.BlockSpec((2, _NODE_BLK, FEAT), lambda i: (0, i, 0)),
            pl.BlockSpec((FEAT, FEAT), lambda i: (0, 0)),
            pl.BlockSpec((1, FEAT), lambda i: (0, 0)),
            pl.BlockSpec((FEAT, FEAT), lambda i: (0, 0)),
            pl.BlockSpec((1, FEAT), lambda i: (0, 0)),
        ],
        out_specs=pl.BlockSpec((_NODE_BLK, FEAT), lambda i: (i, 0)),
        out_shape=jax.ShapeDtypeStruct((NA, FEAT), jnp.float32),
    )(partials, W1, b1.reshape(1, FEAT), W2, b2.reshape(1, FEAT))


def _decoder(ee, Wd1, bd1, Wd2, bd2):
    return pl.pallas_call(
        _decoder_body,
        grid=(B // _DEC_BLK,),
        in_specs=[
            pl.BlockSpec((2, _DEC_BLK, FEAT), lambda i: (0, i, 0)),
            pl.BlockSpec((4 * FEAT, FEAT), lambda i: (0, 0)),
            pl.BlockSpec((1, FEAT), lambda i: (0, 0)),
            pl.BlockSpec((1, FEAT), lambda i: (0, 0)),
            pl.BlockSpec((1, 1), lambda i: (0, 0)),
        ],
        out_specs=pl.BlockSpec((_DEC_BLK, 1), lambda i: (i, 0)),
        out_shape=jax.ShapeDtypeStruct((B, 1), jnp.float32),
    )(ee, Wd1, bd1.reshape(1, FEAT), Wd2.reshape(1, FEAT),
      bd2.reshape(1, 1))


_AR = np.arange(EPAD - E, dtype=np.int32)
_PAD_EDGES = np.stack([N + (_AR % (NA - N)), _AR % N]).astype(np.int32)


def kernel(x, edge_index, curva, idx, W1, b1, W2, b2, Wd1, bd1, Wd2, bd2):
    del curva
    ei = jnp.concatenate([edge_index, jnp.asarray(_PAD_EDGES)],
                         axis=1).reshape(2, NW, CH, C)
    x_p = jnp.pad(x, ((0, NA - N), (0, 0)))

    partials = _edge_scatter(x_p, ei)
    x1 = _node_mlp(partials, W1, b1, W2, b2)

    idx_r = idx.reshape(NW, GCH, GC)
    ents = _pair_gather(x1, idx_r)
    return _decoder(ents.reshape(2, B, FEAT), Wd1, bd1, Wd2, bd2)
